# baseline (speedup 1.0000x reference)
import functools
import math

import jax
import jax.numpy as jnp
import numpy as np
from jax import lax
from jax.experimental import pallas as pl
from jax.experimental.pallas import tpu as pltpu
from jax.experimental.pallas import tpu_sc as plsc

_B, _S, _D, _FF, _E, _TOP_K = 32, 1, 1024, 4096, 8, 2
_LB_COEF = 0.01
_FFT = 1024
_NJ = _FF // _FFT
_L = 16
_NEG = -1e30

_INV_SQRT2 = 1.0 / math.sqrt(2.0)


def _round_bf16(v):
    u = lax.bitcast_convert_type(v, jnp.uint32)
    r = (u + jnp.uint32(0x7FFF) + ((u >> 16) & jnp.uint32(1))) \
        & jnp.uint32(0xFFFF0000)
    return lax.bitcast_convert_type(r, jnp.float32)


def _gate_sc_body(x_hbm, gw_hbm, w_hbm, p_hbm, m_hbm,
                  xv, gwv, wrow, prow, mrow):
    c = lax.axis_index("c")
    s = lax.axis_index("s")
    t = s * 2 + c
    pltpu.sync_copy(x_hbm.at[t], xv)
    pltpu.sync_copy(gw_hbm, gwv)

    iota = lax.iota(jnp.int32, _L)
    logits = jnp.full((_L,), _NEG, dtype=jnp.float32)
    for e in range(_E):
        def body(i, acc):
            xa = _round_bf16(xv[pl.ds(i * _L, _L)])
            ga = _round_bf16(gwv[e, pl.ds(i * _L, _L)])
            return acc + xa * ga
        acc = lax.fori_loop(0, _D // _L, body,
                            jnp.zeros((_L,), jnp.float32))
        se = jnp.sum(acc)
        logits = jnp.where(iota == e, se, logits)

    m1 = jnp.max(logits)
    i1 = jnp.min(jnp.where(logits == m1, iota, _L))
    msk1 = iota == i1
    l2 = jnp.where(msk1, _NEG, logits)
    m2 = jnp.max(l2)
    i2 = jnp.min(jnp.where(l2 == m2, iota, _L))
    msk2 = iota == i2
    pv = jnp.exp(logits - m1)
    pn = pv / jnp.sum(pv)
    msk12 = msk1 | msk2
    s2 = jnp.sum(jnp.where(msk12, pn, 0.0))
    wrow[...] = jnp.where(msk12, pn / s2, 0.0)
    prow[...] = pn
    mrow[...] = jnp.where(msk12, 1.0, 0.0)
    pltpu.sync_copy(wrow, w_hbm.at[t])
    pltpu.sync_copy(prow, p_hbm.at[t])
    pltpu.sync_copy(mrow, m_hbm.at[t])


_gate_sc = functools.partial(
    pl.kernel,
    out_type=[
        jax.ShapeDtypeStruct((_B, _L), jnp.float32),
        jax.ShapeDtypeStruct((_B, _L), jnp.float32),
        jax.ShapeDtypeStruct((_B, _L), jnp.float32),
    ],
    mesh=plsc.VectorSubcoreMesh(core_axis_name="c", subcore_axis_name="s"),
    scratch_types=[
        pltpu.VMEM((_D,), jnp.float32),
        pltpu.VMEM((_E, _D), jnp.float32),
        pltpu.VMEM((_L,), jnp.float32),
        pltpu.VMEM((_L,), jnp.float32),
        pltpu.VMEM((_L,), jnp.float32),
    ],
    compiler_params=pltpu.CompilerParams(needs_layout_passes=False),
)(_gate_sc_body)


def _moe_body(x_ref, w_ref, p_ref, m_ref, fc1w_ref, fc1b_ref, fc2w_ref,
              fc2b_ref, out_ref, aux_ref):
    e = pl.program_id(0)
    j = pl.program_id(1)

    @pl.when((e == 0) & (j == 0))
    def _gate():
        load = jnp.mean(m_ref[...], axis=0, keepdims=True)
        imp = jnp.mean(p_ref[...], axis=0, keepdims=True)
        aux_ref[...] = _LB_COEF * _E * jnp.sum(load * imp, axis=1,
                                               keepdims=True)
        out_ref[...] = jnp.zeros_like(out_ref)

    xb = x_ref[...].astype(jnp.bfloat16)
    h = jax.lax.dot_general(
        xb, fc1w_ref[0].astype(jnp.bfloat16), (((1,), (1,)), ((), ())),
        preferred_element_type=jnp.float32)
    h = h + fc1b_ref[0, 0, 0]
    h = 0.5 * h * (1.0 + jax.lax.erf(h * _INV_SQRT2))
    part = jax.lax.dot_general(
        h.astype(jnp.bfloat16), fc2w_ref[0].astype(jnp.bfloat16),
        (((1,), (1,)), ((), ())),
        preferred_element_type=jnp.float32)

    lane_e = jax.lax.broadcasted_iota(jnp.int32, (_B, _E), 1)
    we = jnp.sum(jnp.where(lane_e == e, w_ref[...], 0.0), axis=1,
                 keepdims=True)
    out_ref[...] += we * part

    @pl.when(j == 0)
    def _bias2():
        out_ref[...] += we * fc2b_ref[0]


@jax.jit
def _moe(x2, gate_w, fc1_w, fc1b_r, fc2_w, fc2b_r):
    wfull, pfull, mfull = _gate_sc(x2, gate_w)
    w8 = wfull[:, :_E]
    p8 = pfull[:, :_E]
    m8 = mfull[:, :_E]
    out, aux = pl.pallas_call(
        _moe_body,
        grid=(_E, _NJ),
        in_specs=[
            pl.BlockSpec((_B, _D), lambda e, j: (0, 0)),
            pl.BlockSpec((_B, _E), lambda e, j: (0, 0)),
            pl.BlockSpec((_B, _E), lambda e, j: (0, 0)),
            pl.BlockSpec((_B, _E), lambda e, j: (0, 0)),
            pl.BlockSpec((1, _FFT, _D), lambda e, j: (e, j, 0)),
            pl.BlockSpec((1, 1, 1, _FFT), lambda e, j: (e, j, 0, 0)),
            pl.BlockSpec((1, _D, _FFT), lambda e, j: (e, 0, j)),
            pl.BlockSpec((1, 1, _D), lambda e, j: (e, 0, 0)),
        ],
        out_specs=[
            pl.BlockSpec((_B, _D), lambda e, j: (0, 0)),
            pl.BlockSpec((1, 1), lambda e, j: (0, 0)),
        ],
        out_shape=[
            jax.ShapeDtypeStruct((_B, _D), jnp.float32),
            jax.ShapeDtypeStruct((1, 1), jnp.float32),
        ],
    )(x2, w8, p8, m8, fc1_w, fc1b_r, fc2_w, fc2b_r)
    return out, aux


def kernel(x, gate_w, fc1_w, fc1_b, fc2_w, fc2_b):
    x2 = x.reshape(_B * _S, _D)
    fc1b_r = fc1_b.reshape(_E, _NJ, 1, _FFT)
    fc2b_r = fc2_b.reshape(_E, 1, _D)
    out, aux = _moe(x2, gate_w, fc1_w, fc1b_r, fc2_w, fc2b_r)
    return out.reshape(_B, _S, _D), aux.reshape(())


# speedup vs baseline: 1.2138x; 1.2138x over previous
import functools
import math

import jax
import jax.numpy as jnp
import numpy as np
from jax.experimental import pallas as pl
from jax.experimental.pallas import tpu as pltpu

_B, _S, _D, _FF, _E, _TOP_K = 32, 1, 1024, 4096, 8, 2
_LB_COEF = 0.01
_FFT = 1024
_NJ = _FF // _FFT

_INV_SQRT2 = 1.0 / math.sqrt(2.0)


def _moe_body(x_ref, gw_ref, fc1w_ref, fc1b_ref, fc2w_ref, fc2b_ref,
              out_ref, aux_ref, w_ref):
    e = pl.program_id(0)
    j = pl.program_id(1)

    @pl.when((e == 0) & (j == 0))
    def _gate():
        xv = x_ref[...]
        logits = jax.lax.dot_general(
            xv, gw_ref[...], (((1,), (1,)), ((), ())),
            preferred_element_type=jnp.float32)
        lane = jax.lax.broadcasted_iota(jnp.int32, logits.shape, 1)
        m1 = jnp.max(logits, axis=1, keepdims=True)
        i1 = jnp.min(jnp.where(logits == m1, lane, _E), axis=1, keepdims=True)
        msk1 = lane == i1
        l2 = jnp.where(msk1, -jnp.inf, logits)
        m2 = jnp.max(l2, axis=1, keepdims=True)
        i2 = jnp.min(jnp.where(l2 == m2, lane, _E), axis=1, keepdims=True)
        msk2 = lane == i2
        b = jnp.exp(m2 - m1)
        denom = 1.0 + b
        w1 = 1.0 / denom
        w2 = b / denom
        w_ref[...] = (jnp.where(msk1, w1, 0.0) + jnp.where(msk2, w2, 0.0))
        p = jnp.exp(logits - m1)
        p = p / jnp.sum(p, axis=1, keepdims=True)
        load = jnp.mean(msk1.astype(jnp.float32) + msk2.astype(jnp.float32),
                        axis=0, keepdims=True)
        imp = jnp.mean(p, axis=0, keepdims=True)
        aux_ref[...] = _LB_COEF * _E * jnp.sum(load * imp, axis=1,
                                               keepdims=True)
        out_ref[...] = jnp.zeros_like(out_ref)

    xb = x_ref[...].astype(jnp.bfloat16)
    h = jax.lax.dot_general(
        xb, fc1w_ref[0].astype(jnp.bfloat16), (((1,), (1,)), ((), ())),
        preferred_element_type=jnp.float32)
    h = h + fc1b_ref[0, 0, 0]
    h = 0.5 * h * (1.0 + jax.lax.erf(h * _INV_SQRT2))
    part = jax.lax.dot_general(
        h.astype(jnp.bfloat16), fc2w_ref[0].astype(jnp.bfloat16),
        (((1,), (1,)), ((), ())),
        preferred_element_type=jnp.float32)

    lane_e = jax.lax.broadcasted_iota(jnp.int32, (_B, _E), 1)
    we = jnp.sum(jnp.where(lane_e == e, w_ref[...], 0.0), axis=1,
                 keepdims=True)
    out_ref[...] += we * part

    @pl.when(j == 0)
    def _bias2():
        out_ref[...] += we * fc2b_ref[0]


@jax.jit
def _moe(x2, gate_w, fc1_w, fc1b_r, fc2_w, fc2b_r):
    out, aux = pl.pallas_call(
        _moe_body,
        grid=(_E, _NJ),
        in_specs=[
            pl.BlockSpec((_B, _D), lambda e, j: (0, 0)),
            pl.BlockSpec((_E, _D), lambda e, j: (0, 0)),
            pl.BlockSpec((1, _FFT, _D), lambda e, j: (e, j, 0)),
            pl.BlockSpec((1, 1, 1, _FFT), lambda e, j: (e, j, 0, 0)),
            pl.BlockSpec((1, _D, _FFT), lambda e, j: (e, 0, j)),
            pl.BlockSpec((1, 1, _D), lambda e, j: (e, 0, 0)),
        ],
        out_specs=[
            pl.BlockSpec((_B, _D), lambda e, j: (0, 0)),
            pl.BlockSpec((1, 1), lambda e, j: (0, 0)),
        ],
        out_shape=[
            jax.ShapeDtypeStruct((_B, _D), jnp.float32),
            jax.ShapeDtypeStruct((1, 1), jnp.float32),
        ],
        scratch_shapes=[pltpu.VMEM((_B, _E), jnp.float32)],
    )(x2, gate_w, fc1_w, fc1b_r, fc2_w, fc2b_r)
    return out, aux


def kernel(x, gate_w, fc1_w, fc1_b, fc2_w, fc2_b):
    x2 = x.reshape(_B * _S, _D)
    fc1b_r = fc1_b.reshape(_E, _NJ, 1, _FFT)
    fc2b_r = fc2_b.reshape(_E, 1, _D)
    out, aux = _moe(x2, gate_w, fc1_w, fc1b_r, fc2_w, fc2b_r)
    return out.reshape(_B, _S, _D), aux.reshape(())
